# baseline (speedup 1.0000x reference)
import jax
import jax.numpy as jnp
from jax import lax
from jax.experimental import pallas as pl
from jax.experimental.pallas import tpu as pltpu
from jax.experimental.pallas import tpu_sc as plsc

N_NODES = 10000
N_PAIRS = 640000
HIDDEN = 768
DIM_EMB = 64

NW = 32
PAIRS_PER_W = N_PAIRS // NW
CHUNK = 4000
NCHUNK = PAIRS_PER_W // CHUNK
STEPS = CHUNK // 16


def _repack_body(idx_ref, o0_ref, o1_ref):
    o0_ref[...] = idx_ref[:, 0]
    o1_ref[...] = idx_ref[:, 1]


def _repack(idx):
    blk = 5120
    return pl.pallas_call(
        _repack_body,
        grid=(N_PAIRS // blk,),
        in_specs=[pl.BlockSpec((blk, 2), lambda i: (i, 0))],
        out_specs=[
            pl.BlockSpec((blk,), lambda i: (i,)),
            pl.BlockSpec((blk,), lambda i: (i,)),
        ],
        out_shape=[
            jax.ShapeDtypeStruct((N_PAIRS,), jnp.int32),
            jax.ShapeDtypeStruct((N_PAIRS,), jnp.int32),
        ],
    )(idx)


def _expand_body(z0_ref, z1_ref, o_ref):
    o_ref[...] = jnp.concatenate(
        [z0_ref[...][:, None], z1_ref[...][:, None]], axis=1)


def _expand(z0, z1):
    blk = 5120
    return pl.pallas_call(
        _expand_body,
        grid=(N_PAIRS // blk,),
        in_specs=[
            pl.BlockSpec((blk,), lambda i: (i,)),
            pl.BlockSpec((blk,), lambda i: (i,)),
        ],
        out_specs=pl.BlockSpec((blk, 2), lambda i: (i, 0)),
        out_shape=jax.ShapeDtypeStruct((N_PAIRS, 2), jnp.float32),
    )(z0, z1)


def _mlp_body(x_ref, w1_ref, b1_ref, w2_ref, b2_ref, wc4_ref, bias4_ref, o_ref):
    h = jnp.maximum(
        jnp.dot(x_ref[...], w1_ref[...], preferred_element_type=jnp.float32)
        + b1_ref[...], 0.0)
    h = jnp.maximum(
        jnp.dot(h, w2_ref[...], preferred_element_type=jnp.float32)
        + b2_ref[...], 0.0)
    o_ref[...] = (
        jnp.dot(h, wc4_ref[...], preferred_element_type=jnp.float32)
        + bias4_ref[...])


def _node_table(x, w1, b1r, w2, b2r, wc4, bias4):
    blk = 1000
    return pl.pallas_call(
        _mlp_body,
        grid=(N_NODES // blk,),
        in_specs=[
            pl.BlockSpec((blk, HIDDEN), lambda i: (i, 0)),
            pl.BlockSpec((HIDDEN, DIM_EMB), lambda i: (0, 0)),
            pl.BlockSpec((1, DIM_EMB), lambda i: (0, 0)),
            pl.BlockSpec((DIM_EMB, DIM_EMB), lambda i: (0, 0)),
            pl.BlockSpec((1, DIM_EMB), lambda i: (0, 0)),
            pl.BlockSpec((DIM_EMB, 4), lambda i: (0, 0)),
            pl.BlockSpec((1, 4), lambda i: (0, 0)),
        ],
        out_specs=pl.BlockSpec((blk, 4), lambda i: (i, 0)),
        out_shape=jax.ShapeDtypeStruct((N_NODES, 4), jnp.float32),
    )(x, w1, b1r, w2, b2r, wc4, bias4)


def _pair_body(tab_hbm, i0_hbm, i1_hbm, z0_hbm, z1_hbm,
               tab_v, i0_v, i1_v, o0_v, o1_v):
    wid = lax.axis_index("s") * 2 + lax.axis_index("c")
    pltpu.sync_copy(tab_hbm, tab_v)
    for c in range(NCHUNK):
        base = wid * PAIRS_PER_W + c * CHUNK
        pltpu.sync_copy(i0_hbm.at[pl.ds(base, CHUNK)], i0_v)
        pltpu.sync_copy(i1_hbm.at[pl.ds(base, CHUNK)], i1_v)

        def step(j, carry):
            s = pl.ds(j * 16, 16)
            a0 = i0_v[s] * 4
            a1 = i1_v[s] * 4
            z0 = plsc.load_gather(tab_v, [a0]) + plsc.load_gather(tab_v, [a1 + 2])
            z1 = plsc.load_gather(tab_v, [a0 + 1]) + plsc.load_gather(tab_v, [a1 + 3])
            o0_v[s] = z0
            o1_v[s] = z1
            return carry

        lax.fori_loop(0, STEPS, step, 0)
        pltpu.sync_copy(o0_v, z0_hbm.at[pl.ds(base, CHUNK)])
        pltpu.sync_copy(o1_v, z1_hbm.at[pl.ds(base, CHUNK)])


def _pair_logits(tab_flat, i0, i1):
    mesh = plsc.VectorSubcoreMesh(core_axis_name="c", subcore_axis_name="s")
    return pl.kernel(
        _pair_body,
        mesh=mesh,
        out_type=(
            jax.ShapeDtypeStruct((N_PAIRS,), jnp.float32),
            jax.ShapeDtypeStruct((N_PAIRS,), jnp.float32),
        ),
        scratch_types=[
            pltpu.VMEM((4 * N_NODES,), jnp.float32),
            pltpu.VMEM((CHUNK,), jnp.int32),
            pltpu.VMEM((CHUNK,), jnp.int32),
            pltpu.VMEM((CHUNK,), jnp.float32),
            pltpu.VMEM((CHUNK,), jnp.float32),
        ],
        compiler_params=pltpu.CompilerParams(needs_layout_passes=False),
    )(tab_flat, i0, i1)


def _loss_body(z0_ref, z1_ref, lab_ref, o_ref, acc_ref):
    i = pl.program_id(0)
    z0 = z0_ref[...]
    z1 = z1_ref[...]
    m = jnp.maximum(z0, z1)
    lse = m + jnp.log(jnp.exp(z0 - m) + jnp.exp(z1 - m))
    zt = jnp.where(lab_ref[...] == 0, z0, z1)
    part = jnp.sum(lse - zt)

    @pl.when(i == 0)
    def _init():
        acc_ref[0] = 0.0

    acc_ref[0] += part

    @pl.when(i == pl.num_programs(0) - 1)
    def _fin():
        o_ref[...] = jnp.broadcast_to(acc_ref[0] / float(N_PAIRS), (1, 1))


def _loss(z02d, z12d, lab2d):
    rows = z02d.shape[0]
    blk = 1000
    return pl.pallas_call(
        _loss_body,
        grid=(rows // blk,),
        in_specs=[
            pl.BlockSpec((blk, 128), lambda i: (i, 0)),
            pl.BlockSpec((blk, 128), lambda i: (i, 0)),
            pl.BlockSpec((blk, 128), lambda i: (i, 0)),
        ],
        out_specs=pl.BlockSpec((1, 1), lambda i: (0, 0)),
        out_shape=jax.ShapeDtypeStruct((1, 1), jnp.float32),
        scratch_shapes=[pltpu.SMEM((1,), jnp.float32)],
    )(z02d, z12d, lab2d)


def kernel(idx, adjacency_matrix, node_embeddings, label, W1, b1, W2, b2, Wc, bc):
    del adjacency_matrix
    wc4 = jnp.concatenate([Wc[:DIM_EMB], Wc[DIM_EMB:]], axis=1)
    bias4 = jnp.concatenate([bc, jnp.zeros((2,), jnp.float32)])[None, :]
    table = _node_table(node_embeddings, W1, b1[None, :], W2, b2[None, :],
                        wc4, bias4)
    i0, i1 = _repack(idx.astype(jnp.int32))
    z0, z1 = _pair_logits(table.reshape(-1), i0, i1)
    rows = N_PAIRS // 128
    loss2d = _loss(z0.reshape(rows, 128), z1.reshape(rows, 128),
                   label.astype(jnp.int32).reshape(rows, 128))
    logits = _expand(z0, z1)
    return (loss2d.reshape(()), logits)


# speedup vs baseline: 1.0520x; 1.0520x over previous
import jax
import jax.numpy as jnp
from jax import lax
from jax.experimental import pallas as pl
from jax.experimental.pallas import tpu as pltpu
from jax.experimental.pallas import tpu_sc as plsc

N_NODES = 10000
N_PAIRS = 640000
HIDDEN = 768
DIM_EMB = 64

NW = 32
PAIRS_PER_W = N_PAIRS // NW
CHUNK = 4000
NCHUNK = PAIRS_PER_W // CHUNK
STEPS = CHUNK // 16


def _mlp_body(x_ref, w1_ref, b1_ref, w2_ref, b2_ref, wc4_ref, bias4_ref, o_ref):
    h = jnp.maximum(
        jnp.dot(x_ref[...], w1_ref[...], preferred_element_type=jnp.float32)
        + b1_ref[...], 0.0)
    h = jnp.maximum(
        jnp.dot(h, w2_ref[...], preferred_element_type=jnp.float32)
        + b2_ref[...], 0.0)
    o_ref[...] = (
        jnp.dot(h, wc4_ref[...], preferred_element_type=jnp.float32)
        + bias4_ref[...])


def _node_table(x, w1, b1r, w2, b2r, wc4, bias4):
    blk = 1000
    return pl.pallas_call(
        _mlp_body,
        grid=(N_NODES // blk,),
        in_specs=[
            pl.BlockSpec((blk, HIDDEN), lambda i: (i, 0)),
            pl.BlockSpec((HIDDEN, DIM_EMB), lambda i: (0, 0)),
            pl.BlockSpec((1, DIM_EMB), lambda i: (0, 0)),
            pl.BlockSpec((DIM_EMB, DIM_EMB), lambda i: (0, 0)),
            pl.BlockSpec((1, DIM_EMB), lambda i: (0, 0)),
            pl.BlockSpec((DIM_EMB, 4), lambda i: (0, 0)),
            pl.BlockSpec((1, 4), lambda i: (0, 0)),
        ],
        out_specs=pl.BlockSpec((blk, 4), lambda i: (i, 0)),
        out_shape=jax.ShapeDtypeStruct((N_NODES, 4), jnp.float32),
    )(x, w1, b1r, w2, b2r, wc4, bias4)


def _pair_body(tab_hbm, idx_hbm, lgt_hbm, z0_hbm, z1_hbm,
               tab_v, idx_v, oc_v, o0_v, o1_v):
    wid = lax.axis_index("s") * 2 + lax.axis_index("c")
    pltpu.sync_copy(tab_hbm, tab_v)
    lanes = lax.iota(jnp.int32, 16)
    zeros = jnp.zeros((16,), jnp.int32)
    ones = zeros + 1
    for c in range(NCHUNK):
        base = wid * PAIRS_PER_W + c * CHUNK
        pltpu.sync_copy(idx_hbm.at[pl.ds(base, CHUNK), :], idx_v)

        def step(j, carry):
            pvec = j * 16 + lanes
            i0 = plsc.load_gather(idx_v, [pvec, zeros])
            i1 = plsc.load_gather(idx_v, [pvec, ones])
            a0 = i0 * 4
            a1 = i1 * 4
            z0 = plsc.load_gather(tab_v, [a0]) + plsc.load_gather(tab_v, [a1 + 2])
            z1 = plsc.load_gather(tab_v, [a0 + 1]) + plsc.load_gather(tab_v, [a1 + 3])
            plsc.store_scatter(oc_v, [pvec, zeros], z0)
            plsc.store_scatter(oc_v, [pvec, ones], z1)
            s = pl.ds(j * 16, 16)
            o0_v[s] = z0
            o1_v[s] = z1
            return carry

        lax.fori_loop(0, STEPS, step, 0)
        pltpu.sync_copy(oc_v, lgt_hbm.at[pl.ds(base, CHUNK), :])
        pltpu.sync_copy(o0_v, z0_hbm.at[pl.ds(base, CHUNK)])
        pltpu.sync_copy(o1_v, z1_hbm.at[pl.ds(base, CHUNK)])


def _pair_logits(tab_flat, idx):
    mesh = plsc.VectorSubcoreMesh(core_axis_name="c", subcore_axis_name="s")
    return pl.kernel(
        _pair_body,
        mesh=mesh,
        out_type=(
            jax.ShapeDtypeStruct((N_PAIRS, 2), jnp.float32),
            jax.ShapeDtypeStruct((N_PAIRS,), jnp.float32),
            jax.ShapeDtypeStruct((N_PAIRS,), jnp.float32),
        ),
        scratch_types=[
            pltpu.VMEM((4 * N_NODES,), jnp.float32),
            pltpu.VMEM((CHUNK, 2), jnp.int32),
            pltpu.VMEM((CHUNK, 2), jnp.float32),
            pltpu.VMEM((CHUNK,), jnp.float32),
            pltpu.VMEM((CHUNK,), jnp.float32),
        ],
        compiler_params=pltpu.CompilerParams(
            needs_layout_passes=False, use_tc_tiling_on_sc=False),
    )(tab_flat, idx)


def _loss_body(z0_ref, z1_ref, lab_ref, o_ref, acc_ref):
    i = pl.program_id(0)
    z0 = z0_ref[...]
    z1 = z1_ref[...]
    m = jnp.maximum(z0, z1)
    lse = m + jnp.log(jnp.exp(z0 - m) + jnp.exp(z1 - m))
    zt = jnp.where(lab_ref[...] == 0, z0, z1)
    part = jnp.sum(lse - zt)

    @pl.when(i == 0)
    def _init():
        acc_ref[0] = 0.0

    acc_ref[0] += part

    @pl.when(i == pl.num_programs(0) - 1)
    def _fin():
        o_ref[...] = jnp.broadcast_to(acc_ref[0] / float(N_PAIRS), (1, 1))


def _loss(z02d, z12d, lab2d):
    rows = z02d.shape[0]
    blk = 1000
    return pl.pallas_call(
        _loss_body,
        grid=(rows // blk,),
        in_specs=[
            pl.BlockSpec((blk, 128), lambda i: (i, 0)),
            pl.BlockSpec((blk, 128), lambda i: (i, 0)),
            pl.BlockSpec((blk, 128), lambda i: (i, 0)),
        ],
        out_specs=pl.BlockSpec((1, 1), lambda i: (0, 0)),
        out_shape=jax.ShapeDtypeStruct((1, 1), jnp.float32),
        scratch_shapes=[pltpu.SMEM((1,), jnp.float32)],
    )(z02d, z12d, lab2d)


def kernel(idx, adjacency_matrix, node_embeddings, label, W1, b1, W2, b2, Wc, bc):
    del adjacency_matrix
    wc4 = jnp.concatenate([Wc[:DIM_EMB], Wc[DIM_EMB:]], axis=1)
    bias4 = jnp.concatenate([bc, jnp.zeros((2,), jnp.float32)])[None, :]
    table = _node_table(node_embeddings, W1, b1[None, :], W2, b2[None, :],
                        wc4, bias4)
    logits, z0, z1 = _pair_logits(table.reshape(-1), idx.astype(jnp.int32))
    rows = N_PAIRS // 128
    loss2d = _loss(z0.reshape(rows, 128), z1.reshape(rows, 128),
                   label.astype(jnp.int32).reshape(rows, 128))
    return (loss2d.reshape(()), logits)


# speedup vs baseline: 11.3420x; 10.7814x over previous
import jax
import jax.numpy as jnp
from jax import lax
from jax.experimental import pallas as pl
from jax.experimental.pallas import tpu as pltpu
from jax.experimental.pallas import tpu_sc as plsc

N_NODES = 10000
N_PAIRS = 640000
HIDDEN = 768
DIM_EMB = 64

NW = 32
PAIRS_PER_W = N_PAIRS // NW
CHUNK = 4000
NCHUNK = PAIRS_PER_W // CHUNK
STEPS = CHUNK // 16


def _mlp_body(x_ref, w1_ref, b1_ref, w2_ref, b2_ref, wc4_ref, bias4_ref, o_ref):
    h = jnp.maximum(
        jnp.dot(x_ref[...], w1_ref[...], preferred_element_type=jnp.float32)
        + b1_ref[...], 0.0)
    h = jnp.maximum(
        jnp.dot(h, w2_ref[...], preferred_element_type=jnp.float32)
        + b2_ref[...], 0.0)
    o_ref[...] = (
        jnp.dot(h, wc4_ref[...], preferred_element_type=jnp.float32)
        + bias4_ref[...])


def _node_table(x, w1, b1r, w2, b2r, wc4, bias4):
    blk = 1000
    return pl.pallas_call(
        _mlp_body,
        grid=(N_NODES // blk,),
        in_specs=[
            pl.BlockSpec((blk, HIDDEN), lambda i: (i, 0)),
            pl.BlockSpec((HIDDEN, DIM_EMB), lambda i: (0, 0)),
            pl.BlockSpec((1, DIM_EMB), lambda i: (0, 0)),
            pl.BlockSpec((DIM_EMB, DIM_EMB), lambda i: (0, 0)),
            pl.BlockSpec((1, DIM_EMB), lambda i: (0, 0)),
            pl.BlockSpec((DIM_EMB, 4), lambda i: (0, 0)),
            pl.BlockSpec((1, 4), lambda i: (0, 0)),
        ],
        out_specs=pl.BlockSpec((blk, 4), lambda i: (i, 0)),
        out_shape=jax.ShapeDtypeStruct((N_NODES, 4), jnp.float32),
    )(x, w1, b1r, w2, b2r, wc4, bias4)


def _pair_body(tab_hbm, i0_hbm, i1_hbm, z0_hbm, z1_hbm,
               tab_v, i0_v, i1_v, o0_v, o1_v):
    wid = lax.axis_index("s") * 2 + lax.axis_index("c")
    pltpu.sync_copy(tab_hbm, tab_v)
    for c in range(NCHUNK):
        base = wid * PAIRS_PER_W + c * CHUNK
        pltpu.sync_copy(i0_hbm.at[pl.ds(base, CHUNK)], i0_v)
        pltpu.sync_copy(i1_hbm.at[pl.ds(base, CHUNK)], i1_v)

        def step(j, carry):
            s = pl.ds(j * 16, 16)
            a0 = i0_v[s] * 4
            a1 = i1_v[s] * 4
            z0 = plsc.load_gather(tab_v, [a0]) + plsc.load_gather(tab_v, [a1 + 2])
            z1 = plsc.load_gather(tab_v, [a0 + 1]) + plsc.load_gather(tab_v, [a1 + 3])
            o0_v[s] = z0
            o1_v[s] = z1
            return carry

        lax.fori_loop(0, STEPS, step, 0)
        pltpu.sync_copy(o0_v, z0_hbm.at[pl.ds(base, CHUNK)])
        pltpu.sync_copy(o1_v, z1_hbm.at[pl.ds(base, CHUNK)])


def _pair_logits(tab_flat, i0, i1):
    mesh = plsc.VectorSubcoreMesh(core_axis_name="c", subcore_axis_name="s")
    return pl.kernel(
        _pair_body,
        mesh=mesh,
        out_type=(
            jax.ShapeDtypeStruct((N_PAIRS,), jnp.float32),
            jax.ShapeDtypeStruct((N_PAIRS,), jnp.float32),
        ),
        scratch_types=[
            pltpu.VMEM((4 * N_NODES,), jnp.float32),
            pltpu.VMEM((CHUNK,), jnp.int32),
            pltpu.VMEM((CHUNK,), jnp.int32),
            pltpu.VMEM((CHUNK,), jnp.float32),
            pltpu.VMEM((CHUNK,), jnp.float32),
        ],
        compiler_params=pltpu.CompilerParams(needs_layout_passes=False),
    )(tab_flat, i0, i1)


def _loss_body(z0_ref, z1_ref, lab_ref, o_ref, acc_ref):
    i = pl.program_id(0)
    z0 = z0_ref[...]
    z1 = z1_ref[...]
    m = jnp.maximum(z0, z1)
    lse = m + jnp.log(jnp.exp(z0 - m) + jnp.exp(z1 - m))
    zt = jnp.where(lab_ref[...] == 0, z0, z1)
    part = jnp.sum(lse - zt)

    @pl.when(i == 0)
    def _init():
        acc_ref[0] = 0.0

    acc_ref[0] += part

    @pl.when(i == pl.num_programs(0) - 1)
    def _fin():
        o_ref[...] = jnp.broadcast_to(acc_ref[0] / float(N_PAIRS), (1, 1))


def _loss(z02d, z12d, lab2d):
    rows = z02d.shape[0]
    blk = 1000
    return pl.pallas_call(
        _loss_body,
        grid=(rows // blk,),
        in_specs=[
            pl.BlockSpec((blk, 128), lambda i: (i, 0)),
            pl.BlockSpec((blk, 128), lambda i: (i, 0)),
            pl.BlockSpec((blk, 128), lambda i: (i, 0)),
        ],
        out_specs=pl.BlockSpec((1, 1), lambda i: (0, 0)),
        out_shape=jax.ShapeDtypeStruct((1, 1), jnp.float32),
        scratch_shapes=[pltpu.SMEM((1,), jnp.float32)],
    )(z02d, z12d, lab2d)


def kernel(idx, adjacency_matrix, node_embeddings, label, W1, b1, W2, b2, Wc, bc):
    del adjacency_matrix
    wc4 = jnp.concatenate([Wc[:DIM_EMB], Wc[DIM_EMB:]], axis=1)
    bias4 = jnp.concatenate([bc, jnp.zeros((2,), jnp.float32)])[None, :]
    table = _node_table(node_embeddings, W1, b1[None, :], W2, b2[None, :],
                        wc4, bias4)
    idx = idx.astype(jnp.int32)
    z0, z1 = _pair_logits(table.reshape(-1), idx[:, 0], idx[:, 1])
    rows = N_PAIRS // 128
    loss2d = _loss(z0.reshape(rows, 128), z1.reshape(rows, 128),
                   label.astype(jnp.int32).reshape(rows, 128))
    logits = jnp.stack([z0, z1], axis=1)
    return (loss2d.reshape(()), logits)
